# baseline (speedup 1.0000x reference)
import jax
import jax.numpy as jnp
from jax import lax
from jax.experimental import pallas as pl
from jax.experimental.pallas import tpu as pltpu
from jax.experimental.pallas import tpu_sc as plsc

_VOCAB = 50257
_BATCH = 4
_SEQ = 4096
_K = 2
_NC = 2
_NS = 16
_NW = _NC * _NS
_L = 16
_CHUNK = 128
_NSB = _SEQ // _CHUNK


def _router_body(ids_hbm, table_hbm, out_hbm, ids_v, idx1_v, g_v, sem):
    wid = lax.axis_index("s") * _NC + lax.axis_index("c")
    b = wid // (_NSB // _BATCH)
    sbase = (wid % (_NSB // _BATCH)) * _BATCH
    pltpu.sync_copy(ids_hbm.at[pl.ds(sbase, _BATCH), b], ids_v)
    copies = [
        pltpu.async_copy(table_hbm.at[ids_v.at[j]], g_v.at[j, 0], sem)
        for j in range(_BATCH)
    ]
    for j in range(_BATCH):
        for g in range(_CHUNK // _L):
            sl = pl.ds(g * _L, _L)
            idx1_v[j, sl] = ids_v[j, sl] + _VOCAB
    copies += [
        pltpu.async_copy(table_hbm.at[idx1_v.at[j]], g_v.at[j, 1], sem)
        for j in range(_BATCH)
    ]
    for c in copies:
        c.wait()
    pltpu.sync_copy(g_v, out_hbm.at[b, pl.ds(sbase, _BATCH)])


@jax.jit
def _route(ids3, table_flat):
    mesh = plsc.VectorSubcoreMesh(
        core_axis_name="c", subcore_axis_name="s", num_cores=_NC,
        num_subcores=_NS,
    )
    call = pl.kernel(
        _router_body,
        out_type=jax.ShapeDtypeStruct((_BATCH, _NSB, _K, _CHUNK), jnp.int32),
        mesh=mesh,
        scratch_types=[
            pltpu.VMEM((_BATCH, _CHUNK), jnp.int32),
            pltpu.VMEM((_BATCH, _CHUNK), jnp.int32),
            pltpu.VMEM((_BATCH, _K, _CHUNK), jnp.int32),
            pltpu.SemaphoreType.DMA,
        ],
        compiler_params=pltpu.CompilerParams(
            use_tc_tiling_on_sc=False, needs_layout_passes=False,
        ),
    )
    return call(ids3, table_flat)


def kernel(input, hash_table):
    ids3 = input.astype(jnp.int32).reshape(_BATCH, _NSB, _CHUNK).transpose(1, 0, 2)
    table_flat = hash_table.T.reshape(-1)
    out = _route(ids3, table_flat)
    return out.transpose(0, 1, 3, 2).reshape(_BATCH, _SEQ, _K)


# speedup vs baseline: 1.0043x; 1.0043x over previous
import jax
import jax.numpy as jnp
from jax import lax
from jax.experimental import pallas as pl
from jax.experimental.pallas import tpu as pltpu
from jax.experimental.pallas import tpu_sc as plsc

_VOCAB = 50257
_BATCH = 4
_SEQ = 4096
_K = 2
_NC = 2
_NS = 16
_NW = _NC * _NS
_L = 16
_CHUNK = 128
_NSB = _SEQ // _CHUNK


def _router_body(ids_hbm, table_hbm, out_hbm, ids_v, idx1_v, g0_v, g1_v,
                 sem, wsem):
    wid = lax.axis_index("s") * _NC + lax.axis_index("c")
    pltpu.sync_copy(ids_hbm.at[wid], ids_v)
    copies = [
        pltpu.async_copy(table_hbm.at[ids_v.at[j]], g0_v.at[j], sem)
        for j in range(_BATCH)
    ]
    for j in range(_BATCH):
        for g in range(_CHUNK // _L):
            sl = pl.ds(g * _L, _L)
            idx1_v[j, sl] = ids_v[j, sl] + _VOCAB
    copies += [
        pltpu.async_copy(table_hbm.at[idx1_v.at[j]], g1_v.at[j], sem)
        for j in range(_BATCH)
    ]
    for c in copies:
        c.wait()
    w0 = pltpu.async_copy(g0_v, out_hbm.at[:, wid, 0], wsem)
    w1 = pltpu.async_copy(g1_v, out_hbm.at[:, wid, 1], wsem)
    w0.wait()
    w1.wait()


@jax.jit
def _route(ids3, table_flat):
    mesh = plsc.VectorSubcoreMesh(
        core_axis_name="c", subcore_axis_name="s", num_cores=_NC,
        num_subcores=_NS,
    )
    call = pl.kernel(
        _router_body,
        out_type=jax.ShapeDtypeStruct((_BATCH, _NSB, _K, _CHUNK), jnp.int32),
        mesh=mesh,
        scratch_types=[
            pltpu.VMEM((_BATCH, _CHUNK), jnp.int32),
            pltpu.VMEM((_BATCH, _CHUNK), jnp.int32),
            pltpu.VMEM((_BATCH, _CHUNK), jnp.int32),
            pltpu.VMEM((_BATCH, _CHUNK), jnp.int32),
            pltpu.SemaphoreType.DMA,
            pltpu.SemaphoreType.DMA,
        ],
        compiler_params=pltpu.CompilerParams(
            use_tc_tiling_on_sc=False, needs_layout_passes=False,
        ),
    )
    return call(ids3, table_flat)


def kernel(input, hash_table):
    ids3 = input.astype(jnp.int32).reshape(_BATCH, _NSB, _CHUNK).transpose(1, 0, 2)
    table_flat = hash_table.T.reshape(-1)
    out = _route(ids3, table_flat)
    return out.transpose(0, 1, 3, 2).reshape(_BATCH, _SEQ, _K)
